# baseline (speedup 1.0000x reference)
import functools

import jax
import jax.numpy as jnp
from jax import lax
from jax.experimental import pallas as pl
from jax.experimental.pallas import tpu as pltpu
from jax.experimental.pallas import tpu_sc as plsc

EMB_DIM = 64
BATCH = 4096
HIST = 200
HIST_PAD = 208
NC = 2
NS = 16
NW = NC * NS
ROWS_PER_W = BATCH // NW
K = 4
NG = ROWS_PER_W // K
OUT_PAD = 16
TC_BLK = 512


def _sc_sum_kernel(tokens_hbm, emb_hbm, out_hbm,
                   tok0, tok1, rows0, rows1, out0, out1,
                   sg0, sg1, st0, st1, so0, so1):
    wid = lax.axis_index("s") * NC + lax.axis_index("c")
    row0 = wid * ROWS_PER_W
    toks = (tok0, tok1)
    rows = (rows0, rows1)
    outs = (out0, out1)
    sgs = (sg0, sg1)
    sts = (st0, st1)
    sos = (so0, so1)

    def fire_gathers(g, buf):
        pltpu.async_copy(emb_hbm.at[toks[buf]], rows[buf], sgs[buf])

    def drain_gathers(buf):
        pltpu.make_async_copy(
            emb_hbm.at[pl.ds(0, K * HIST_PAD)], rows[buf], sgs[buf]).wait()

    def fire_tokens(g, buf):
        pltpu.async_copy(
            tokens_hbm.at[wid * NG + g], toks[buf], sts[buf])

    def drain_tokens(buf):
        pltpu.make_async_copy(
            tokens_hbm.at[0], toks[buf], sts[buf]).wait()

    def fire_out(g, buf):
        pltpu.async_copy(outs[buf],
                         out_hbm.at[pl.ds(row0 + g * K, K)], sos[buf])

    def drain_out(buf):
        pltpu.make_async_copy(outs[buf],
                              out_hbm.at[pl.ds(0, K)], sos[buf]).wait()

    def sum_group(buf):
        rv = rows[buf]
        for r in range(K):
            def sum_body(i, acc):
                rb = r * HIST_PAD + 2 * i
                return (acc[0] + rv[rb, pl.ds(0, 16)]
                        + rv[rb + 1, pl.ds(0, 16)],
                        acc[1] + rv[rb, pl.ds(16, 16)]
                        + rv[rb + 1, pl.ds(16, 16)],
                        acc[2] + rv[rb, pl.ds(32, 16)]
                        + rv[rb + 1, pl.ds(32, 16)],
                        acc[3] + rv[rb, pl.ds(48, 16)]
                        + rv[rb + 1, pl.ds(48, 16)])

            z = jnp.zeros((16,), jnp.float32)
            acc = lax.fori_loop(0, HIST_PAD // 2, sum_body, (z, z, z, z),
                                unroll=4)
            for c in range(4):
                outs[buf][r, pl.ds(16 * c, 16)] = acc[c]

    fire_tokens(0, 0)
    drain_tokens(0)
    fire_gathers(0, 0)
    fire_tokens(1, 1)
    drain_tokens(1)
    fire_gathers(1, 1)

    def pair_body(gg, carry):
        for buf in range(2):
            g = 2 * gg + buf
            drain_gathers(buf)
            @pl.when(g + 2 < NG)
            def _():
                fire_tokens(g + 2, buf)
            @pl.when(g >= 2)
            def _():
                drain_out(buf)
            sum_group(buf)
            fire_out(g, buf)
            @pl.when(g + 2 < NG)
            def _():
                drain_tokens(buf)
                fire_gathers(g + 2, buf)
        return carry

    lax.fori_loop(0, NG // 2, pair_body, 0)
    drain_out(0)
    drain_out(1)


def _tc_finish_kernel(tokens_ref, sums_ref, e0_ref, wt_ref, b_ref, out_ref):
    nz = jnp.sum((tokens_ref[...] == 0).astype(jnp.float32), axis=1,
                 keepdims=True)
    s = sums_ref[...] - nz * e0_ref[...]
    inv = 1.0 / (jnp.float32(HIST_PAD) - nz)
    dots = jnp.dot(s, wt_ref[...], preferred_element_type=jnp.float32)
    out_ref[...] = dots * inv + b_ref[...]


@jax.jit
def _run(tokens_p, emb, wt_pad, b_pad, e0):
    mesh = plsc.VectorSubcoreMesh(core_axis_name="c", subcore_axis_name="s",
                                  num_cores=NC, num_subcores=NS)
    sums = pl.kernel(
        _sc_sum_kernel,
        out_type=jax.ShapeDtypeStruct((BATCH, EMB_DIM), jnp.float32),
        mesh=mesh,
        scratch_types=[
            pltpu.VMEM((K * HIST_PAD,), jnp.int32),
            pltpu.VMEM((K * HIST_PAD,), jnp.int32),
            pltpu.VMEM((K * HIST_PAD, EMB_DIM), jnp.float32),
            pltpu.VMEM((K * HIST_PAD, EMB_DIM), jnp.float32),
            pltpu.VMEM((K, EMB_DIM), jnp.float32),
            pltpu.VMEM((K, EMB_DIM), jnp.float32),
            pltpu.SemaphoreType.DMA,
            pltpu.SemaphoreType.DMA,
            pltpu.SemaphoreType.DMA,
            pltpu.SemaphoreType.DMA,
            pltpu.SemaphoreType.DMA,
            pltpu.SemaphoreType.DMA,
        ],
        compiler_params=pltpu.CompilerParams(use_tc_tiling_on_sc=False),
    )(tokens_p.reshape(BATCH // K, K * HIST_PAD), emb)

    out = pl.pallas_call(
        _tc_finish_kernel,
        out_shape=jax.ShapeDtypeStruct((BATCH, OUT_PAD), jnp.float32),
        grid=(BATCH // TC_BLK,),
        in_specs=[
            pl.BlockSpec((TC_BLK, HIST_PAD), lambda i: (i, 0)),
            pl.BlockSpec((TC_BLK, EMB_DIM), lambda i: (i, 0)),
            pl.BlockSpec((1, EMB_DIM), lambda i: (0, 0)),
            pl.BlockSpec((EMB_DIM, OUT_PAD), lambda i: (0, 0)),
            pl.BlockSpec((1, OUT_PAD), lambda i: (0, 0)),
        ],
        out_specs=pl.BlockSpec((TC_BLK, OUT_PAD), lambda i: (i, 0)),
    )(tokens_p, sums, e0, wt_pad, b_pad)
    return out


def kernel(tokens, emb, W, b):
    tokens_p = jnp.pad(tokens.astype(jnp.int32), ((0, 0), (0, HIST_PAD - HIST)))
    wt_pad = jnp.pad(W, ((0, OUT_PAD - W.shape[0]), (0, 0))).T
    b_pad = jnp.pad(b, (0, OUT_PAD - b.shape[0]))[None, :]
    e0 = emb[0][None, :]
    out = _run(tokens_p, emb, wt_pad, b_pad, e0)
    return out[:, :5]


# speedup vs baseline: 1.9216x; 1.9216x over previous
import functools

import jax
import jax.numpy as jnp
from jax import lax
from jax.experimental import pallas as pl
from jax.experimental.pallas import tpu as pltpu
from jax.experimental.pallas import tpu_sc as plsc

EMB_DIM = 64
BATCH = 4096
HIST = 200
NC = 2
NS = 16
NW = NC * NS
ROWS_PER_W = BATCH // NW
K = 4
NG = ROWS_PER_W // K
OUT_PAD = 16
TC_BLK = 512


def _sc_sum_kernel(tokens_hbm, emb_hbm, out_hbm,
                   tok0, tok1, rows0, rows1, out0, out1,
                   sg0, sg1, st0, st1, so0, so1):
    wid = lax.axis_index("s") * NC + lax.axis_index("c")
    row0 = wid * ROWS_PER_W
    toks = (tok0, tok1)
    rows = (rows0, rows1)
    outs = (out0, out1)
    sgs = (sg0, sg1)
    sts = (st0, st1)
    sos = (so0, so1)

    def fire_gathers(g, buf):
        pltpu.async_copy(emb_hbm.at[toks[buf]], rows[buf], sgs[buf])

    def drain_gathers(buf):
        pltpu.make_async_copy(
            emb_hbm.at[pl.ds(0, K * HIST)], rows[buf], sgs[buf]).wait()

    def fire_tokens(g, buf):
        pltpu.async_copy(
            tokens_hbm.at[wid * NG + g], toks[buf], sts[buf])

    def drain_tokens(buf):
        pltpu.make_async_copy(
            tokens_hbm.at[0], toks[buf], sts[buf]).wait()

    def fire_out(g, buf):
        pltpu.async_copy(outs[buf],
                         out_hbm.at[pl.ds(row0 + g * K, K)], sos[buf])

    def drain_out(buf):
        pltpu.make_async_copy(outs[buf],
                              out_hbm.at[pl.ds(0, K)], sos[buf]).wait()

    def sum_group(buf):
        rv = rows[buf]
        for r in range(K):
            def sum_body(i, acc):
                rb = r * HIST + 2 * i
                return (acc[0] + rv[rb, pl.ds(0, 16)]
                        + rv[rb + 1, pl.ds(0, 16)],
                        acc[1] + rv[rb, pl.ds(16, 16)]
                        + rv[rb + 1, pl.ds(16, 16)],
                        acc[2] + rv[rb, pl.ds(32, 16)]
                        + rv[rb + 1, pl.ds(32, 16)],
                        acc[3] + rv[rb, pl.ds(48, 16)]
                        + rv[rb + 1, pl.ds(48, 16)])

            z = jnp.zeros((16,), jnp.float32)
            acc = lax.fori_loop(0, HIST // 2, sum_body, (z, z, z, z),
                                unroll=4)
            for c in range(4):
                outs[buf][r, pl.ds(16 * c, 16)] = acc[c]

    fire_tokens(0, 0)
    drain_tokens(0)
    fire_gathers(0, 0)
    fire_tokens(1, 1)
    drain_tokens(1)
    fire_gathers(1, 1)

    def pair_body(gg, carry):
        for buf in range(2):
            g = 2 * gg + buf
            drain_gathers(buf)
            @pl.when(g + 2 < NG)
            def _():
                fire_tokens(g + 2, buf)
            @pl.when(g >= 2)
            def _():
                drain_out(buf)
            sum_group(buf)
            fire_out(g, buf)
            @pl.when(g + 2 < NG)
            def _():
                drain_tokens(buf)
                fire_gathers(g + 2, buf)
        return carry

    lax.fori_loop(0, NG // 2, pair_body, 0)
    drain_out(0)
    drain_out(1)


def _tc_finish_kernel(tokens_ref, sums_ref, e0_ref, wt_ref, b_ref, out_ref):
    nz = jnp.sum((tokens_ref[...] == 0).astype(jnp.float32), axis=1,
                 keepdims=True)
    s = sums_ref[...] - nz * e0_ref[...]
    inv = 1.0 / (jnp.float32(HIST) - nz)
    dots = jnp.dot(s, wt_ref[...], preferred_element_type=jnp.float32)
    out_ref[...] = dots * inv + b_ref[...]


@jax.jit
def _run(tokens_i, emb, wt_pad, b_pad, e0):
    mesh = plsc.VectorSubcoreMesh(core_axis_name="c", subcore_axis_name="s",
                                  num_cores=NC, num_subcores=NS)
    sums = pl.kernel(
        _sc_sum_kernel,
        out_type=jax.ShapeDtypeStruct((BATCH, EMB_DIM), jnp.float32),
        mesh=mesh,
        scratch_types=[
            pltpu.VMEM((K * HIST,), jnp.int32),
            pltpu.VMEM((K * HIST,), jnp.int32),
            pltpu.VMEM((K * HIST, EMB_DIM), jnp.float32),
            pltpu.VMEM((K * HIST, EMB_DIM), jnp.float32),
            pltpu.VMEM((K, EMB_DIM), jnp.float32),
            pltpu.VMEM((K, EMB_DIM), jnp.float32),
            pltpu.SemaphoreType.DMA,
            pltpu.SemaphoreType.DMA,
            pltpu.SemaphoreType.DMA,
            pltpu.SemaphoreType.DMA,
            pltpu.SemaphoreType.DMA,
            pltpu.SemaphoreType.DMA,
        ],
        compiler_params=pltpu.CompilerParams(use_tc_tiling_on_sc=False),
    )(tokens_i.reshape(BATCH // K, K * HIST), emb)

    out = pl.pallas_call(
        _tc_finish_kernel,
        out_shape=jax.ShapeDtypeStruct((BATCH, OUT_PAD), jnp.float32),
        grid=(BATCH // TC_BLK,),
        in_specs=[
            pl.BlockSpec((TC_BLK, HIST), lambda i: (i, 0)),
            pl.BlockSpec((TC_BLK, EMB_DIM), lambda i: (i, 0)),
            pl.BlockSpec((1, EMB_DIM), lambda i: (0, 0)),
            pl.BlockSpec((EMB_DIM, OUT_PAD), lambda i: (0, 0)),
            pl.BlockSpec((1, OUT_PAD), lambda i: (0, 0)),
        ],
        out_specs=pl.BlockSpec((TC_BLK, OUT_PAD), lambda i: (i, 0)),
    )(tokens_i, sums, e0, wt_pad, b_pad)
    return out


def kernel(tokens, emb, W, b):
    tokens_i = tokens.astype(jnp.int32)
    wt_pad = jnp.pad(W, ((0, OUT_PAD - W.shape[0]), (0, 0))).T
    b_pad = jnp.pad(b, (0, OUT_PAD - b.shape[0]))[None, :]
    e0 = emb[0][None, :]
    out = _run(tokens_i, emb, wt_pad, b_pad, e0)
    return out[:, :5]


# speedup vs baseline: 1.9247x; 1.0016x over previous
import functools

import jax
import jax.numpy as jnp
from jax import lax
from jax.experimental import pallas as pl
from jax.experimental.pallas import tpu as pltpu
from jax.experimental.pallas import tpu_sc as plsc

EMB_DIM = 64
BATCH = 4096
HIST = 200
NC = 2
NS = 16
NW = NC * NS
ROWS_PER_W = BATCH // NW
K = 4
NG = ROWS_PER_W // K
OUT_PAD = 16
TC_BLK = 512


def _sc_sum_kernel(tokens_hbm, emb_hbm, out_hbm,
                   tok0, tok1, rows0, rows1, out0, out1,
                   sg0, sg1, st0, st1, so0, so1):
    wid = lax.axis_index("s") * NC + lax.axis_index("c")
    row0 = wid * ROWS_PER_W
    toks = (tok0, tok1)
    rows = (rows0, rows1)
    outs = (out0, out1)
    sgs = (sg0, sg1)
    sts = (st0, st1)
    sos = (so0, so1)

    def fire_gathers(g, buf):
        for r in range(K):
            pltpu.async_copy(emb_hbm.at[toks[buf].at[r]],
                             rows[buf].at[pl.ds(r * HIST, HIST)], sgs[buf])

    def drain_gathers(buf):
        pltpu.make_async_copy(
            emb_hbm.at[pl.ds(0, K * HIST)], rows[buf], sgs[buf]).wait()

    def fire_tokens(g, buf):
        pltpu.async_copy(
            tokens_hbm.at[pl.ds(row0 + g * K, K)], toks[buf], sts[buf])

    def drain_tokens(buf):
        pltpu.make_async_copy(
            tokens_hbm.at[pl.ds(0, K)], toks[buf], sts[buf]).wait()

    def fire_out(g, buf):
        pltpu.async_copy(outs[buf],
                         out_hbm.at[pl.ds(row0 + g * K, K)], sos[buf])

    def drain_out(buf):
        pltpu.make_async_copy(outs[buf],
                              out_hbm.at[pl.ds(0, K)], sos[buf]).wait()

    def sum_group(buf):
        rv = rows[buf]
        for r in range(K):
            def sum_body(i, acc):
                rb = r * HIST + 2 * i
                return (acc[0] + rv[rb, pl.ds(0, 16)]
                        + rv[rb + 1, pl.ds(0, 16)],
                        acc[1] + rv[rb, pl.ds(16, 16)]
                        + rv[rb + 1, pl.ds(16, 16)],
                        acc[2] + rv[rb, pl.ds(32, 16)]
                        + rv[rb + 1, pl.ds(32, 16)],
                        acc[3] + rv[rb, pl.ds(48, 16)]
                        + rv[rb + 1, pl.ds(48, 16)])

            z = jnp.zeros((16,), jnp.float32)
            acc = lax.fori_loop(0, HIST // 2, sum_body, (z, z, z, z),
                                unroll=4)
            for c in range(4):
                outs[buf][r, pl.ds(16 * c, 16)] = acc[c]

    fire_tokens(0, 0)
    drain_tokens(0)
    fire_gathers(0, 0)
    fire_tokens(1, 1)
    drain_tokens(1)
    fire_gathers(1, 1)

    def pair_body(gg, carry):
        for buf in range(2):
            g = 2 * gg + buf
            drain_gathers(buf)
            @pl.when(g + 2 < NG)
            def _():
                fire_tokens(g + 2, buf)
            @pl.when(g >= 2)
            def _():
                drain_out(buf)
            sum_group(buf)
            fire_out(g, buf)
            @pl.when(g + 2 < NG)
            def _():
                drain_tokens(buf)
                fire_gathers(g + 2, buf)
        return carry

    lax.fori_loop(0, NG // 2, pair_body, 0)
    drain_out(0)
    drain_out(1)


def _tc_finish_kernel(tokens_ref, sums_ref, e0_ref, wt_ref, b_ref, out_ref):
    nz = jnp.sum((tokens_ref[...] == 0).astype(jnp.float32), axis=1,
                 keepdims=True)
    s = sums_ref[...] - nz * e0_ref[...]
    inv = 1.0 / (jnp.float32(HIST) - nz)
    dots = jnp.dot(s, wt_ref[...], preferred_element_type=jnp.float32)
    out_ref[...] = dots * inv + b_ref[...]


@jax.jit
def _run(tokens_i, emb, wt_pad, b_pad, e0):
    mesh = plsc.VectorSubcoreMesh(core_axis_name="c", subcore_axis_name="s",
                                  num_cores=NC, num_subcores=NS)
    sums = pl.kernel(
        _sc_sum_kernel,
        out_type=jax.ShapeDtypeStruct((BATCH, EMB_DIM), jnp.float32),
        mesh=mesh,
        scratch_types=[
            pltpu.VMEM((K, HIST), jnp.int32),
            pltpu.VMEM((K, HIST), jnp.int32),
            pltpu.VMEM((K * HIST, EMB_DIM), jnp.float32),
            pltpu.VMEM((K * HIST, EMB_DIM), jnp.float32),
            pltpu.VMEM((K, EMB_DIM), jnp.float32),
            pltpu.VMEM((K, EMB_DIM), jnp.float32),
            pltpu.SemaphoreType.DMA,
            pltpu.SemaphoreType.DMA,
            pltpu.SemaphoreType.DMA,
            pltpu.SemaphoreType.DMA,
            pltpu.SemaphoreType.DMA,
            pltpu.SemaphoreType.DMA,
        ],
        compiler_params=pltpu.CompilerParams(use_tc_tiling_on_sc=False),
    )(tokens_i, emb)

    out = pl.pallas_call(
        _tc_finish_kernel,
        out_shape=jax.ShapeDtypeStruct((BATCH, OUT_PAD), jnp.float32),
        grid=(BATCH // TC_BLK,),
        in_specs=[
            pl.BlockSpec((TC_BLK, HIST), lambda i: (i, 0)),
            pl.BlockSpec((TC_BLK, EMB_DIM), lambda i: (i, 0)),
            pl.BlockSpec((1, EMB_DIM), lambda i: (0, 0)),
            pl.BlockSpec((EMB_DIM, OUT_PAD), lambda i: (0, 0)),
            pl.BlockSpec((1, OUT_PAD), lambda i: (0, 0)),
        ],
        out_specs=pl.BlockSpec((TC_BLK, OUT_PAD), lambda i: (i, 0)),
    )(tokens_i, sums, e0, wt_pad, b_pad)
    return out


def kernel(tokens, emb, W, b):
    tokens_i = tokens.astype(jnp.int32)
    wt_pad = jnp.pad(W, ((0, OUT_PAD - W.shape[0]), (0, 0))).T
    b_pad = jnp.pad(b, (0, OUT_PAD - b.shape[0]))[None, :]
    e0 = emb[0][None, :]
    out = _run(tokens_i, emb, wt_pad, b_pad, e0)
    return out[:, :5]
